# Initial kernel scaffold; baseline (speedup 1.0000x reference)
import jax
import jax.numpy as jnp
from jax import lax
from jax.experimental import pallas as pl
from jax.experimental.pallas import tpu as pltpu
from jax.experimental.pallas import tpu_sc as plsc

_K = 1000
_D = 128
_B = 4096
_L = 200

_NC = 2
_NS = 16
_NW = _NC * _NS

_N = _B * _L
_PER_W = _N // _NW
_SUB = 128
_NSUB = _PER_W // _SUB


def _weight_body(r_ref, l_ref, h_ref, e_ref, w_ref):
    r = r_ref[...]
    w_ref[...] = r * l_ref[...] + (1.0 - r) * h_ref[...] + e_ref[...]


def _gather_body(w_hbm, idx_hbm, out_hbm, idx_v, rows_v, sem):
    wid = lax.axis_index("s") * _NC + lax.axis_index("c")
    base = wid * _NSUB
    pltpu.sync_copy(idx_hbm.at[pl.ds(base, _NSUB)], idx_v)

    def sub(j, carry):
        pltpu.async_copy(w_hbm.at[idx_v.at[j]], rows_v, sem).wait()
        pltpu.sync_copy(rows_v, out_hbm.at[pl.ds((base + j) * _SUB, _SUB)])
        return carry

    lax.fori_loop(0, _NSUB, sub, 0)


@jax.jit
def kernel(idx, E, l, h, r):
    weight = pl.pallas_call(
        _weight_body,
        out_shape=jax.ShapeDtypeStruct((_K, _D), jnp.float32),
    )(r.reshape(_K, 1), l.reshape(1, _D), h.reshape(1, _D), E)

    idx2 = idx.reshape(_N // _SUB, _SUB).astype(jnp.int32)

    gather = pl.kernel(
        _gather_body,
        out_type=jax.ShapeDtypeStruct((_N, _D), jnp.float32),
        mesh=plsc.VectorSubcoreMesh(core_axis_name="c", subcore_axis_name="s"),
        scratch_types=[
            pltpu.VMEM((_NSUB, _SUB), jnp.int32),
            pltpu.VMEM((_SUB, _D), jnp.float32),
            pltpu.SemaphoreType.DMA,
        ],
    )
    out = gather(weight, idx2)
    return out.reshape(_B, _L, _D)


# speedup vs baseline: 5.6448x; 5.6448x over previous
import jax
import jax.numpy as jnp
from jax import lax
from jax.experimental import pallas as pl
from jax.experimental.pallas import tpu as pltpu
from jax.experimental.pallas import tpu_sc as plsc

_K = 1000
_D = 128
_B = 4096
_L = 200

_NC = 2
_NS = 16
_NW = _NC * _NS

_N = _B * _L
_PER_W = _N // _NW
_SUB = 128
_NSUB = _PER_W // _SUB


def _weight_body(r_ref, l_ref, h_ref, e_ref, w_ref):
    r = r_ref[...]
    w_ref[...] = r * l_ref[...] + (1.0 - r) * h_ref[...] + e_ref[...]


def _gather_body(w_hbm, idx_hbm, out_hbm, idx_v, rows_v, sem):
    wid = lax.axis_index("s") * _NC + lax.axis_index("c")
    base = wid * _NSUB
    pltpu.sync_copy(idx_hbm.at[pl.ds(base, _NSUB)], idx_v)

    def sub(j, carry):
        pltpu.async_copy(w_hbm.at[idx_v.at[j]], rows_v, sem).wait()
        pltpu.sync_copy(rows_v, out_hbm.at[pl.ds((base + j) * _SUB, _SUB)])
        return carry

    lax.fori_loop(0, _NSUB, sub, 0)


@jax.jit
def kernel(idx, E, l, h, r):
    weight = pl.pallas_call(
        _weight_body,
        out_shape=jax.ShapeDtypeStruct((_K, _D), jnp.float32),
    )(r.reshape(_K, 1), l.reshape(1, _D), h.reshape(1, _D), E)

    idx2 = idx.reshape(_N // _SUB, _SUB).astype(jnp.int32)

    gather = pl.kernel(
        _gather_body,
        out_type=jax.ShapeDtypeStruct((_N, _D), jnp.float32),
        mesh=plsc.VectorSubcoreMesh(
            core_axis_name="c", subcore_axis_name="s",
            num_cores=_NC, num_subcores=_NS,
        ),
        scratch_types=[
            pltpu.VMEM((_NSUB, _SUB), jnp.int32),
            pltpu.VMEM((_SUB, _D), jnp.float32),
            pltpu.SemaphoreType.DMA,
        ],
    )
    out = gather(weight, idx2)
    return out.reshape(_B, _L, _D)


# speedup vs baseline: 9.6366x; 1.7072x over previous
import jax
import jax.numpy as jnp
from jax import lax
from jax.experimental import pallas as pl
from jax.experimental.pallas import tpu as pltpu
from jax.experimental.pallas import tpu_sc as plsc

_K = 1000
_D = 128
_B = 4096
_L = 200

_NC = 2
_NS = 16
_NW = _NC * _NS

_N = _B * _L
_PER_W = _N // _NW
_SUB = 128
_NSUB = _PER_W // _SUB


def _weight_body(r_ref, l_ref, h_ref, e_ref, w_ref):
    r = r_ref[...]
    w_ref[...] = r * l_ref[...] + (1.0 - r) * h_ref[...] + e_ref[...]


def _gather_body(w_hbm, idx_hbm, out_hbm, table_sh, idx_v, rows_v, sem):
    sid = lax.axis_index("s")
    wid = sid * _NC + lax.axis_index("c")
    base = wid * _NSUB

    @pl.when(sid == 0)
    def _():
        pltpu.sync_copy(w_hbm, table_sh)

    plsc.subcore_barrier()

    pltpu.sync_copy(idx_hbm.at[pl.ds(base, _NSUB)], idx_v)

    def sub(j, carry):
        pltpu.async_copy(table_sh.at[idx_v.at[j]], rows_v, sem).wait()
        pltpu.sync_copy(rows_v, out_hbm.at[pl.ds((base + j) * _SUB, _SUB)])
        return carry

    lax.fori_loop(0, _NSUB, sub, 0)


@jax.jit
def kernel(idx, E, l, h, r):
    weight = pl.pallas_call(
        _weight_body,
        out_shape=jax.ShapeDtypeStruct((_K, _D), jnp.float32),
    )(r.reshape(_K, 1), l.reshape(1, _D), h.reshape(1, _D), E)

    idx2 = idx.reshape(_N // _SUB, _SUB).astype(jnp.int32)

    gather = pl.kernel(
        _gather_body,
        out_type=jax.ShapeDtypeStruct((_N, _D), jnp.float32),
        mesh=plsc.VectorSubcoreMesh(
            core_axis_name="c", subcore_axis_name="s",
            num_cores=_NC, num_subcores=_NS,
        ),
        scratch_types=[
            pltpu.VMEM_SHARED((_K, _D), jnp.float32),
            pltpu.VMEM((_NSUB, _SUB), jnp.int32),
            pltpu.VMEM((_SUB, _D), jnp.float32),
            pltpu.SemaphoreType.DMA,
        ],
    )
    out = gather(weight, idx2)
    return out.reshape(_B, _L, _D)


# speedup vs baseline: 14.1764x; 1.4711x over previous
import jax
import jax.numpy as jnp
from jax import lax
from jax.experimental import pallas as pl
from jax.experimental.pallas import tpu as pltpu
from jax.experimental.pallas import tpu_sc as plsc

_K = 1000
_D = 128
_B = 4096
_L = 200

_NC = 2
_NS = 16
_NW = _NC * _NS

_N = _B * _L
_PER_W = _N // _NW
_SUB = 128
_NSUB = _PER_W // _SUB
_NBUF = 4


def _weight_body(r_ref, l_ref, h_ref, e_ref, w_ref):
    r = r_ref[...]
    w_ref[...] = r * l_ref[...] + (1.0 - r) * h_ref[...] + e_ref[...]


def _gather_body(w_hbm, idx_hbm, out_hbm, table_sh, idx_v, rows_v, semg, semo):
    sid = lax.axis_index("s")
    wid = sid * _NC + lax.axis_index("c")
    base = wid * _NSUB

    @pl.when(sid == 0)
    def _():
        pltpu.sync_copy(w_hbm, table_sh)

    plsc.subcore_barrier()

    pltpu.sync_copy(idx_hbm.at[pl.ds(base, _NSUB)], idx_v)

    def group(p, carry):
        @pl.when(p > 0)
        def _():
            for b in range(_NBUF):
                pltpu.make_async_copy(
                    rows_v.at[b], out_hbm.at[pl.ds(0, _SUB)], semo
                ).wait()

        gathers = []
        for b in range(_NBUF):
            c = p * _NBUF + b
            gathers.append(
                pltpu.async_copy(table_sh.at[idx_v.at[c]], rows_v.at[b], semg)
            )
        for b in range(_NBUF):
            c = p * _NBUF + b
            gathers[b].wait()
            pltpu.async_copy(
                rows_v.at[b], out_hbm.at[pl.ds((base + c) * _SUB, _SUB)], semo
            )
        return carry

    lax.fori_loop(0, _NSUB // _NBUF, group, 0)

    for b in range(_NBUF):
        pltpu.make_async_copy(
            rows_v.at[b], out_hbm.at[pl.ds(0, _SUB)], semo
        ).wait()


@jax.jit
def kernel(idx, E, l, h, r):
    weight = pl.pallas_call(
        _weight_body,
        out_shape=jax.ShapeDtypeStruct((_K, _D), jnp.float32),
    )(r.reshape(_K, 1), l.reshape(1, _D), h.reshape(1, _D), E)

    idx2 = idx.reshape(_N // _SUB, _SUB).astype(jnp.int32)

    gather = pl.kernel(
        _gather_body,
        out_type=jax.ShapeDtypeStruct((_N, _D), jnp.float32),
        mesh=plsc.VectorSubcoreMesh(
            core_axis_name="c", subcore_axis_name="s",
            num_cores=_NC, num_subcores=_NS,
        ),
        scratch_types=[
            pltpu.VMEM_SHARED((_K, _D), jnp.float32),
            pltpu.VMEM((_NSUB, _SUB), jnp.int32),
            pltpu.VMEM((_NBUF, _SUB, _D), jnp.float32),
            pltpu.SemaphoreType.DMA,
            pltpu.SemaphoreType.DMA,
        ],
    )
    out = gather(weight, idx2)
    return out.reshape(_B, _L, _D)


# speedup vs baseline: 15.4850x; 1.0923x over previous
import jax
import jax.numpy as jnp
from jax import lax
from jax.experimental import pallas as pl
from jax.experimental.pallas import tpu as pltpu
from jax.experimental.pallas import tpu_sc as plsc

_K = 1000
_D = 128
_B = 4096
_L = 200

_NC = 2
_NS = 16
_NW = _NC * _NS

_N = _B * _L
_PER_W = _N // _NW
_SUB = 128
_NSUB = _PER_W // _SUB
_NBUF = 4


def _weight_body(r_ref, l_ref, h_ref, e_ref, w_ref):
    r = r_ref[...]
    w_ref[...] = r * l_ref[...] + (1.0 - r) * h_ref[...] + e_ref[...]


def _gather_body(w_hbm, idx_hbm, out_hbm, table_sh, idx_v, rows_v, semg, semo):
    sid = lax.axis_index("s")
    wid = sid * _NC + lax.axis_index("c")
    base = wid * _NSUB

    @pl.when(sid < 5)
    def _():
        pltpu.sync_copy(
            w_hbm.at[pl.ds(sid * 200, 200)], table_sh.at[pl.ds(sid * 200, 200)]
        )

    plsc.subcore_barrier()

    pltpu.sync_copy(idx_hbm.at[pl.ds(base, _NSUB)], idx_v)

    def group(p, carry):
        gathers = []
        for b in range(_NBUF):
            c = p * _NBUF + b

            @pl.when(p > 0)
            def _(b=b):
                pltpu.make_async_copy(
                    rows_v.at[b], out_hbm.at[pl.ds(0, _SUB)], semo.at[b]
                ).wait()

            gathers.append(
                pltpu.async_copy(table_sh.at[idx_v.at[c]], rows_v.at[b], semg)
            )
        for b in range(_NBUF):
            c = p * _NBUF + b
            gathers[b].wait()
            pltpu.async_copy(
                rows_v.at[b],
                out_hbm.at[pl.ds((base + c) * _SUB, _SUB)],
                semo.at[b],
            )
        return carry

    lax.fori_loop(0, _NSUB // _NBUF, group, 0)

    for b in range(_NBUF):
        pltpu.make_async_copy(
            rows_v.at[b], out_hbm.at[pl.ds(0, _SUB)], semo.at[b]
        ).wait()


@jax.jit
def kernel(idx, E, l, h, r):
    weight = pl.pallas_call(
        _weight_body,
        out_shape=jax.ShapeDtypeStruct((_K, _D), jnp.float32),
    )(r.reshape(_K, 1), l.reshape(1, _D), h.reshape(1, _D), E)

    idx2 = idx.reshape(_N // _SUB, _SUB).astype(jnp.int32)

    gather = pl.kernel(
        _gather_body,
        out_type=jax.ShapeDtypeStruct((_N, _D), jnp.float32),
        mesh=plsc.VectorSubcoreMesh(
            core_axis_name="c", subcore_axis_name="s",
            num_cores=_NC, num_subcores=_NS,
        ),
        scratch_types=[
            pltpu.VMEM_SHARED((_K, _D), jnp.float32),
            pltpu.VMEM((_NSUB, _SUB), jnp.int32),
            pltpu.VMEM((_NBUF, _SUB, _D), jnp.float32),
            pltpu.SemaphoreType.DMA,
            pltpu.SemaphoreType.DMA((_NBUF,)),
        ],
    )
    out = gather(weight, idx2)
    return out.reshape(_B, _L, _D)


# speedup vs baseline: 15.5348x; 1.0032x over previous
import jax
import jax.numpy as jnp
from jax import lax
from jax.experimental import pallas as pl
from jax.experimental.pallas import tpu as pltpu
from jax.experimental.pallas import tpu_sc as plsc

_K = 1000
_D = 128
_B = 4096
_L = 200

_NC = 2
_NS = 16
_NW = _NC * _NS

_N = _B * _L
_PER_W = _N // _NW
_SUB = 128
_NSUB = _PER_W // _SUB
_NBUF = 5


def _weight_body(r_ref, l_ref, h_ref, e_ref, w_ref):
    r = r_ref[...]
    w_ref[...] = r * l_ref[...] + (1.0 - r) * h_ref[...] + e_ref[...]


def _gather_body(w_hbm, idx_hbm, out_hbm, table_sh, idx_v, rows_v, semg, semo):
    sid = lax.axis_index("s")
    wid = sid * _NC + lax.axis_index("c")
    base = wid * _NSUB

    @pl.when(sid < 5)
    def _():
        pltpu.sync_copy(
            w_hbm.at[pl.ds(sid * 200, 200)], table_sh.at[pl.ds(sid * 200, 200)]
        )

    plsc.subcore_barrier()

    pltpu.sync_copy(idx_hbm.at[pl.ds(base, _NSUB)], idx_v)

    def group(p, carry):
        gathers = []
        for b in range(_NBUF):
            c = p * _NBUF + b

            @pl.when(p > 0)
            def _(b=b):
                pltpu.make_async_copy(
                    rows_v.at[b], out_hbm.at[pl.ds(0, _SUB)], semo.at[b]
                ).wait()

            gathers.append(
                pltpu.async_copy(table_sh.at[idx_v.at[c]], rows_v.at[b], semg)
            )
        for b in range(_NBUF):
            c = p * _NBUF + b
            gathers[b].wait()
            pltpu.async_copy(
                rows_v.at[b],
                out_hbm.at[pl.ds((base + c) * _SUB, _SUB)],
                semo.at[b],
            )
        return carry

    lax.fori_loop(0, _NSUB // _NBUF, group, 0)

    for b in range(_NBUF):
        pltpu.make_async_copy(
            rows_v.at[b], out_hbm.at[pl.ds(0, _SUB)], semo.at[b]
        ).wait()


@jax.jit
def kernel(idx, E, l, h, r):
    weight = pl.pallas_call(
        _weight_body,
        out_shape=jax.ShapeDtypeStruct((_K, _D), jnp.float32),
    )(r.reshape(_K, 1), l.reshape(1, _D), h.reshape(1, _D), E)

    idx2 = idx.reshape(_N // _SUB, _SUB).astype(jnp.int32)

    gather = pl.kernel(
        _gather_body,
        out_type=jax.ShapeDtypeStruct((_N, _D), jnp.float32),
        mesh=plsc.VectorSubcoreMesh(
            core_axis_name="c", subcore_axis_name="s",
            num_cores=_NC, num_subcores=_NS,
        ),
        scratch_types=[
            pltpu.VMEM_SHARED((_K, _D), jnp.float32),
            pltpu.VMEM((_NSUB, _SUB), jnp.int32),
            pltpu.VMEM((_NBUF, _SUB, _D), jnp.float32),
            pltpu.SemaphoreType.DMA,
            pltpu.SemaphoreType.DMA((_NBUF,)),
        ],
    )
    out = gather(weight, idx2)
    return out.reshape(_B, _L, _D)
